# baseline (speedup 1.0000x reference)
import jax
import jax.numpy as jnp
from jax.experimental import pallas as pl
from jax.experimental.pallas import tpu as pltpu

_ALPHA = 0.2
_BLK = 1024
_MASKED = -1e4


def _gat_kernel(x_ref, adj_ref, w_ref, a_ref, out_ref, h_s, fneg_s, gneg_s):
    i = pl.program_id(0)
    Bb, N, in_f = x_ref.shape
    out_f = w_ref.shape[1]

    @pl.when(i == 0)
    def _prologue():
        x2 = x_ref[...].reshape(Bb * N, in_f)
        h = jnp.dot(x2, w_ref[...], preferred_element_type=jnp.float32)
        h = jnp.where(jnp.isnan(h), 0.0, h)
        h_s[...] = h
        a = a_ref[...]
        na1 = -a[:, :out_f]
        na2 = -a[:, out_f:]
        fneg_s[...] = jax.lax.dot_general(h, na1, (((1,), (1,)), ((), ())),
                                          preferred_element_type=jnp.float32)
        gneg_s[...] = jax.lax.dot_general(na2, h, (((1,), (1,)), ((), ())),
                                          preferred_element_type=jnp.float32)

    adj_blk = adj_ref[...]
    for b in range(Bb):
        fneg = fneg_s[pl.ds(b * N + i * _BLK, _BLK), :]
        gneg = gneg_s[:, pl.ds(b * N, N)]
        t = fneg + gneg
        t = jnp.where(adj_blk != 0, t, _MASKED)
        e = jnp.exp(jnp.minimum(t, _ALPHA * t))

        rowsum = jnp.sum(e, axis=1, keepdims=True)
        recip = jnp.where(rowsum != 0, 1.0 / rowsum, 1.0)

        h_b = h_s[pl.ds(b * N, N), :]
        hp = jnp.dot(e, h_b, preferred_element_type=jnp.float32) * recip
        hp = jnp.where(jnp.isnan(hp), 0.0, hp)
        hp = jnp.where(hp > 0, hp, jnp.exp(jnp.minimum(hp, 0.0)) - 1.0)
        out_ref[b, :, :] = hp


def kernel(inputBatch, adj, W, a):
    Bb, N, in_f = inputBatch.shape
    out_f = W.shape[1]
    nb = N // _BLK
    return pl.pallas_call(
        _gat_kernel,
        grid=(nb,),
        in_specs=[
            pl.BlockSpec((Bb, N, in_f), lambda i: (0, 0, 0)),
            pl.BlockSpec((_BLK, N), lambda i: (i, 0)),
            pl.BlockSpec((in_f, out_f), lambda i: (0, 0)),
            pl.BlockSpec((1, 2 * out_f), lambda i: (0, 0)),
        ],
        out_specs=pl.BlockSpec((Bb, _BLK, out_f), lambda i: (0, i, 0)),
        out_shape=jax.ShapeDtypeStruct((Bb, N, out_f), jnp.float32),
        scratch_shapes=[
            pltpu.VMEM((Bb * N, out_f), jnp.float32),
            pltpu.VMEM((Bb * N, 1), jnp.float32),
            pltpu.VMEM((1, Bb * N), jnp.float32),
        ],
    )(inputBatch, adj, W, a)


# speedup vs baseline: 1.0373x; 1.0373x over previous
import jax
import jax.numpy as jnp
from jax.experimental import pallas as pl
from jax.experimental.pallas import tpu as pltpu

_ALPHA = 0.2
_BLK = 512
_MASKED = -1e4


def _gat_kernel(x_ref, adj_ref, w_ref, a_ref, out_ref, h_s, fneg_s, gneg_s):
    i = pl.program_id(0)
    Bb, N, in_f = x_ref.shape
    out_f = w_ref.shape[1]

    @pl.when(i == 0)
    def _prologue():
        x2 = x_ref[...].reshape(Bb * N, in_f)
        h = jnp.dot(x2, w_ref[...], preferred_element_type=jnp.float32)
        h = jnp.where(jnp.isnan(h), 0.0, h)
        h_s[...] = h
        a = a_ref[...]
        na1 = -a[:, :out_f]
        na2 = -a[:, out_f:]
        fneg_s[...] = jax.lax.dot_general(h, na1, (((1,), (1,)), ((), ())),
                                          preferred_element_type=jnp.float32)
        gneg_s[...] = jax.lax.dot_general(na2, h, (((1,), (1,)), ((), ())),
                                          preferred_element_type=jnp.float32)

    adj_blk = adj_ref[...]
    for b in range(Bb):
        fneg = fneg_s[pl.ds(b * N + i * _BLK, _BLK), :]
        gneg = gneg_s[:, pl.ds(b * N, N)]
        t = fneg + gneg
        e = adj_blk * jnp.exp(jnp.minimum(t, _ALPHA * t))

        rowsum = jnp.sum(e, axis=1, keepdims=True)
        recip = jnp.where(rowsum != 0, 1.0 / rowsum, 1.0)

        h_b = h_s[pl.ds(b * N, N), :]
        hp = jnp.dot(e.astype(jnp.bfloat16), h_b.astype(jnp.bfloat16),
                     preferred_element_type=jnp.float32) * recip
        hp = jnp.where(jnp.isnan(hp), 0.0, hp)
        hp = jnp.where(hp > 0, hp, jnp.exp(jnp.minimum(hp, 0.0)) - 1.0)
        out_ref[b, :, :] = hp


def kernel(inputBatch, adj, W, a):
    Bb, N, in_f = inputBatch.shape
    out_f = W.shape[1]
    nb = N // _BLK
    return pl.pallas_call(
        _gat_kernel,
        grid=(nb,),
        in_specs=[
            pl.BlockSpec((Bb, N, in_f), lambda i: (0, 0, 0)),
            pl.BlockSpec((_BLK, N), lambda i: (i, 0)),
            pl.BlockSpec((in_f, out_f), lambda i: (0, 0)),
            pl.BlockSpec((1, 2 * out_f), lambda i: (0, 0)),
        ],
        out_specs=pl.BlockSpec((Bb, _BLK, out_f), lambda i: (0, i, 0)),
        out_shape=jax.ShapeDtypeStruct((Bb, N, out_f), jnp.float32),
        scratch_shapes=[
            pltpu.VMEM((Bb * N, out_f), jnp.float32),
            pltpu.VMEM((Bb * N, 1), jnp.float32),
            pltpu.VMEM((1, Bb * N), jnp.float32),
        ],
    )(inputBatch, adj, W, a)


# speedup vs baseline: 1.1611x; 1.1194x over previous
import jax
import jax.numpy as jnp
from jax.experimental import pallas as pl
from jax.experimental.pallas import tpu as pltpu

_ALPHA = 0.2
_BLK = 512
_MASKED = -1e4


def _gat_kernel(x_ref, adj_ref, w_ref, a_ref, out_ref, h_s, fneg_s, gneg_s):
    i = pl.program_id(0)
    Bb, N, in_f = x_ref.shape
    out_f = w_ref.shape[1]

    @pl.when(i == 0)
    def _prologue():
        x2 = x_ref[...].reshape(Bb * N, in_f)
        h = jnp.dot(x2, w_ref[...], preferred_element_type=jnp.float32)
        h = jnp.where(jnp.isnan(h), 0.0, h)
        a = a_ref[...]
        na1 = -a[:, :out_f]
        na2 = -a[:, out_f:]
        fneg_s[...] = jax.lax.dot_general(h, na1, (((1,), (1,)), ((), ())),
                                          preferred_element_type=jnp.float32)
        gneg_s[...] = jax.lax.dot_general(na2, h, (((1,), (1,)), ((), ())),
                                          preferred_element_type=jnp.float32)
        ones = jnp.ones((Bb * N, 8), jnp.float32)
        h_s[...] = jnp.concatenate([h, ones], axis=1).astype(jnp.bfloat16)

    adj_blk = adj_ref[...]
    for b in range(Bb):
        fneg = fneg_s[pl.ds(b * N + i * _BLK, _BLK), :]
        gneg = gneg_s[:, pl.ds(b * N, N)]
        t = fneg + gneg
        e = adj_blk * jnp.exp(jnp.minimum(t, _ALPHA * t))

        h_b = h_s[pl.ds(b * N, N), :]
        p = jnp.dot(e.astype(jnp.bfloat16), h_b,
                    preferred_element_type=jnp.float32)
        rowsum = p[:, out_f:out_f + 1]
        recip = jnp.where(rowsum != 0, 1.0 / rowsum, 1.0)
        hp = p[:, :out_f] * recip
        hp = jnp.where(jnp.isnan(hp), 0.0, hp)
        hp = jnp.where(hp > 0, hp, jnp.exp(jnp.minimum(hp, 0.0)) - 1.0)
        out_ref[b, :, :] = hp


def kernel(inputBatch, adj, W, a):
    Bb, N, in_f = inputBatch.shape
    out_f = W.shape[1]
    nb = N // _BLK
    return pl.pallas_call(
        _gat_kernel,
        grid=(nb,),
        in_specs=[
            pl.BlockSpec((Bb, N, in_f), lambda i: (0, 0, 0)),
            pl.BlockSpec((_BLK, N), lambda i: (i, 0)),
            pl.BlockSpec((in_f, out_f), lambda i: (0, 0)),
            pl.BlockSpec((1, 2 * out_f), lambda i: (0, 0)),
        ],
        out_specs=pl.BlockSpec((Bb, _BLK, out_f), lambda i: (0, i, 0)),
        out_shape=jax.ShapeDtypeStruct((Bb, N, out_f), jnp.float32),
        scratch_shapes=[
            pltpu.VMEM((Bb * N, out_f + 8), jnp.bfloat16),
            pltpu.VMEM((Bb * N, 1), jnp.float32),
            pltpu.VMEM((1, Bb * N), jnp.float32),
        ],
    )(inputBatch, adj, W, a)
